# Initial kernel scaffold; baseline (speedup 1.0000x reference)
import jax
import jax.numpy as jnp
from jax.experimental import pallas as pl


def kernel(logits, targets):
    raise NotImplementedError("write your pallas kernel here")


# speedup vs baseline: 1.0156x; 1.0156x over previous
import jax
import jax.numpy as jnp
from jax.experimental import pallas as pl
from jax.experimental.pallas import tpu as pltpu


def _ohem_kernel(logits_ref, targets_ref, out_ref,
                 bg_vec, fg_sum, nr_fg, nr_bg):
    i = pl.program_id(0)
    nblk = pl.num_programs(0)
    B, C = logits_ref.shape

    @pl.when(i == 0)
    def _init():
        fg_sum[0] = jnp.float32(0.0)
        nr_fg[0] = jnp.int32(0)
        nr_bg[0] = jnp.int32(0)

    x = logits_ref[...]
    t = targets_ref[...]
    cls = jax.lax.broadcasted_iota(jnp.int32, (1, C), 1) + 1
    z = (t[:, None] == cls).astype(x.dtype)
    le = (jnp.maximum(x, 0.0) - x * z
          + jnp.log(1.0 + jnp.exp(-jnp.abs(x))))
    row = jnp.sum(le, axis=1)
    is_bg = t == 0
    is_fg = t > 0
    bg_vec[pl.ds(i * B, B)] = jnp.where(is_bg, row, 0.0)
    fg_sum[0] += jnp.sum(jnp.where(is_fg, row, 0.0))
    nr_fg[0] += jnp.sum(is_fg.astype(jnp.int32))
    nr_bg[0] += jnp.sum(is_bg.astype(jnp.int32))

    @pl.when(i == nblk - 1)
    def _finish():
        k = jnp.maximum(jnp.int32(128),
                        jnp.minimum(nr_bg[0], nr_fg[0] * 3))

        def body(j, prefix):
            cand = prefix | jax.lax.shift_left(jnp.int32(1), 30 - j)
            candf = jax.lax.bitcast_convert_type(cand, jnp.float32)
            cnt = jnp.sum((bg_vec[...] >= candf).astype(jnp.int32))
            return jnp.where(cnt >= k, cand, prefix)

        prefix = jax.lax.fori_loop(0, 31, body, jnp.int32(0))
        thr = jax.lax.bitcast_convert_type(prefix, jnp.float32)
        v = bg_vec[...]
        gt = v > thr
        cnt_gt = jnp.sum(gt.astype(jnp.int32))
        sum_gt = jnp.sum(jnp.where(gt, v, 0.0))
        train_bg = sum_gt + (k - cnt_gt).astype(jnp.float32) * thr
        out_ref[0] = (fg_sum[0] + train_bg) * 0.25


def kernel(logits, targets):
    N, C = logits.shape
    B = 8192
    nblk = N // B
    out = pl.pallas_call(
        _ohem_kernel,
        grid=(nblk,),
        in_specs=[
            pl.BlockSpec((B, C), lambda i: (i, 0)),
            pl.BlockSpec((B,), lambda i: (i,)),
        ],
        out_specs=pl.BlockSpec(memory_space=pltpu.SMEM),
        out_shape=jax.ShapeDtypeStruct((1,), jnp.float32),
        scratch_shapes=[
            pltpu.VMEM((N,), jnp.float32),
            pltpu.SMEM((1,), jnp.float32),
            pltpu.SMEM((1,), jnp.int32),
            pltpu.SMEM((1,), jnp.int32),
        ],
        compiler_params=pltpu.CompilerParams(
            dimension_semantics=("arbitrary",),
        ),
    )(logits, targets)
    return out[0]


# speedup vs baseline: 1.1531x; 1.1354x over previous
import jax
import jax.numpy as jnp
from jax.experimental import pallas as pl
from jax.experimental.pallas import tpu as pltpu

_C = 80
_RPB = 16
_LANES = _C * _RPB


def _ohem_kernel(x_ref, t_ref, out_ref,
                 s_vm, st_vm, cls_vm, rs_vm, fg_sum, nr_fg, nr_bg):
    i = pl.program_id(0)
    nblk = pl.num_programs(0)
    B = x_ref.shape[0]

    @pl.when(i == 0)
    def _init():
        fg_sum[0] = jnp.float32(0.0)
        nr_fg[0] = jnp.int32(0)
        nr_bg[0] = jnp.int32(0)
        jj = jax.lax.broadcasted_iota(jnp.int32, (_LANES, _RPB), 0)
        kk = jax.lax.broadcasted_iota(jnp.int32, (_LANES, _RPB), 1)
        b = jj - _C * kk
        s_vm[...] = ((b >= 0) & (b < _C)).astype(jnp.float32)
        kk2 = jax.lax.broadcasted_iota(jnp.int32, (_RPB, _LANES), 0)
        jj2 = jax.lax.broadcasted_iota(jnp.int32, (_RPB, _LANES), 1)
        b2 = jj2 - _C * kk2
        st_vm[...] = ((b2 >= 0) & (b2 < _C)).astype(jnp.float32)
        j1 = jax.lax.broadcasted_iota(jnp.int32, (1, _LANES), 1)
        seg = jnp.zeros((1, _LANES), jnp.int32)
        for m in range(1, _RPB):
            seg = seg + (j1 >= _C * m).astype(jnp.int32)
        cls_vm[...] = (j1 - _C * seg + 1).astype(jnp.float32)

    x = x_ref[...]
    t3 = t_ref[...]
    t_b = jnp.dot(t3.astype(jnp.float32), st_vm[...],
                  preferred_element_type=jnp.float32)
    sp = (jnp.maximum(x, 0.0)
          + jnp.log(1.0 + jnp.exp(-jnp.abs(x))))
    le = sp - jnp.where(t_b == cls_vm[...], x, 0.0)
    rs16 = jnp.dot(le, s_vm[...],
                   preferred_element_type=jnp.float32)
    is_fg = t3 > 0
    is_bg = t3 == 0
    bg16 = jnp.where(is_bg, rs16, 0.0)
    for c in range(8):
        rs_vm[pl.ds(i * (B // 8), B // 8), c * _RPB:(c + 1) * _RPB] = \
            bg16[c * (B // 8):(c + 1) * (B // 8), :]
    fg_sum[0] += jnp.sum(jnp.where(is_fg, rs16, 0.0))
    nr_fg[0] += jnp.sum(is_fg.astype(jnp.int32))
    nr_bg[0] += jnp.sum(is_bg.astype(jnp.int32))

    @pl.when(i == nblk - 1)
    def _finish():
        k = jnp.maximum(jnp.int32(128),
                        jnp.minimum(nr_bg[0], nr_fg[0] * 3))

        def body(j, prefix):
            cand = prefix | jax.lax.shift_left(jnp.int32(1), 30 - j)
            candf = jax.lax.bitcast_convert_type(cand, jnp.float32)
            cnt = jnp.sum((rs_vm[...] >= candf).astype(jnp.int32))
            return jnp.where(cnt >= k, cand, prefix)

        prefix = jax.lax.fori_loop(0, 31, body, jnp.int32(0))
        thr = jax.lax.bitcast_convert_type(prefix, jnp.float32)
        v = rs_vm[...]
        gt = v > thr
        cnt_gt = jnp.sum(gt.astype(jnp.int32))
        sum_gt = jnp.sum(jnp.where(gt, v, 0.0))
        train_bg = sum_gt + (k - cnt_gt).astype(jnp.float32) * thr
        out_ref[0] = (fg_sum[0] + train_bg) * 0.25


def kernel(logits, targets):
    N, C = logits.shape
    nrow = N * C // _LANES
    x2 = logits.reshape(nrow, _LANES)
    t2 = targets.reshape(nrow, _RPB)
    B = 512
    nblk = nrow // B
    out = pl.pallas_call(
        _ohem_kernel,
        grid=(nblk,),
        in_specs=[
            pl.BlockSpec((B, _LANES), lambda i: (i, 0)),
            pl.BlockSpec((B, _RPB), lambda i: (i, 0)),
        ],
        out_specs=pl.BlockSpec(memory_space=pltpu.SMEM),
        out_shape=jax.ShapeDtypeStruct((1,), jnp.float32),
        scratch_shapes=[
            pltpu.VMEM((_LANES, _RPB), jnp.float32),
            pltpu.VMEM((_RPB, _LANES), jnp.float32),
            pltpu.VMEM((1, _LANES), jnp.float32),
            pltpu.VMEM((N // 128, 128), jnp.float32),
            pltpu.SMEM((1,), jnp.float32),
            pltpu.SMEM((1,), jnp.int32),
            pltpu.SMEM((1,), jnp.int32),
        ],
        compiler_params=pltpu.CompilerParams(
            dimension_semantics=("arbitrary",),
        ),
    )(x2, t2)
    return out[0]


# speedup vs baseline: 1.6167x; 1.4020x over previous
import jax
import jax.numpy as jnp
from jax.experimental import pallas as pl
from jax.experimental.pallas import tpu as pltpu

_C = 80


def _ohem_kernel(x_ref, t_ref, out_ref, rs_vm, tot_sum, nr_fg, nr_bg):
    i = pl.program_id(0)
    nblk = pl.num_programs(0)
    B = x_ref.shape[0]

    @pl.when(i == 0)
    def _init():
        tot_sum[0] = jnp.float32(0.0)
        nr_fg[0] = jnp.int32(0)
        nr_bg[0] = jnp.int32(0)

    x = x_ref[...]
    t = t_ref[...]
    cls = jax.lax.broadcasted_iota(jnp.int32, (1, 1, _C), 2) + 1
    z = t[:, :, None] == cls
    sp = (jnp.maximum(x, 0.0)
          + jnp.log(1.0 + jnp.exp(-jnp.abs(x))))
    le = sp - jnp.where(z, x, 0.0)
    rs = jnp.sum(le, axis=2)
    is_bg = t == 0
    rs_vm[pl.ds(i * B, B), :] = jnp.where(is_bg, rs, 0.0)
    tot_sum[0] += jnp.sum(rs)
    nr_fg[0] += jnp.sum((t > 0).astype(jnp.int32))
    nr_bg[0] += jnp.sum(is_bg.astype(jnp.int32))

    @pl.when(i == nblk - 1)
    def _finish():
        k = jnp.maximum(jnp.int32(128),
                        jnp.minimum(nr_bg[0], nr_fg[0] * 3))

        def body(j, prefix):
            cand = prefix | jax.lax.shift_left(jnp.int32(1), 30 - j)
            candf = jax.lax.bitcast_convert_type(cand, jnp.float32)
            cnt = jnp.sum((rs_vm[...] >= candf).astype(jnp.int32))
            return jnp.where(cnt >= k, cand, prefix)

        prefix = jax.lax.fori_loop(0, 31, body, jnp.int32(0))
        thr = jax.lax.bitcast_convert_type(prefix, jnp.float32)
        v = rs_vm[...]
        gt = v > thr
        cnt_gt = jnp.sum(gt.astype(jnp.int32))
        sum_gt = jnp.sum(jnp.where(gt, v, 0.0))
        train_bg = sum_gt + (k - cnt_gt).astype(jnp.float32) * thr
        bg_total = jnp.sum(v)
        fg_sum = tot_sum[0] - bg_total
        out_ref[0] = (fg_sum + train_bg) * 0.25


def kernel(logits, targets):
    N, C = logits.shape
    x3 = logits.reshape(N // 128, 128, C)
    t2 = targets.reshape(N // 128, 128)
    B = 64
    nblk = (N // 128) // B
    out = pl.pallas_call(
        _ohem_kernel,
        grid=(nblk,),
        in_specs=[
            pl.BlockSpec((B, 128, C), lambda i: (i, 0, 0)),
            pl.BlockSpec((B, 128), lambda i: (i, 0)),
        ],
        out_specs=pl.BlockSpec(memory_space=pltpu.SMEM),
        out_shape=jax.ShapeDtypeStruct((1,), jnp.float32),
        scratch_shapes=[
            pltpu.VMEM((N // 128, 128), jnp.float32),
            pltpu.SMEM((1,), jnp.float32),
            pltpu.SMEM((1,), jnp.int32),
            pltpu.SMEM((1,), jnp.int32),
        ],
        compiler_params=pltpu.CompilerParams(
            dimension_semantics=("arbitrary",),
        ),
    )(x3, t2)
    return out[0]


# speedup vs baseline: 1.7359x; 1.0737x over previous
import jax
import jax.numpy as jnp
from jax.experimental import pallas as pl
from jax.experimental.pallas import tpu as pltpu

_C = 80


def _ohem_kernel(x_ref, t_ref, out_ref, rs_vm, tot_sum, nr_fg, nr_bg):
    i = pl.program_id(0)
    nblk = pl.num_programs(0)
    B = t_ref.shape[0]

    @pl.when(i == 0)
    def _init():
        tot_sum[0] = jnp.float32(0.0)
        nr_fg[0] = jnp.int32(0)
        nr_bg[0] = jnp.int32(0)

    x = x_ref[...].reshape(B, 128, _C)
    t = t_ref[...]
    cls = jax.lax.broadcasted_iota(jnp.int32, (1, 1, _C), 2) + 1
    z = t[:, :, None] == cls
    sp = (jnp.maximum(x, 0.0)
          + jnp.log(1.0 + jnp.exp(-jnp.abs(x))))
    le = sp - jnp.where(z, x, 0.0)
    rs = jnp.sum(le, axis=2)
    is_bg = t == 0
    rs_vm[pl.ds(i * B, B), :] = jnp.where(is_bg, rs, 0.0)
    tot_sum[0] += jnp.sum(rs)
    nr_fg[0] += jnp.sum((t > 0).astype(jnp.int32))
    nr_bg[0] += jnp.sum(is_bg.astype(jnp.int32))

    @pl.when(i == nblk - 1)
    def _finish():
        k = jnp.maximum(jnp.int32(128),
                        jnp.minimum(nr_bg[0], nr_fg[0] * 3))

        def body(j, prefix):
            cand = prefix | jax.lax.shift_left(jnp.int32(1), 30 - j)
            candf = jax.lax.bitcast_convert_type(cand, jnp.float32)
            cnt = jnp.sum((rs_vm[...] >= candf).astype(jnp.int32))
            return jnp.where(cnt >= k, cand, prefix)

        prefix = jax.lax.fori_loop(0, 31, body, jnp.int32(0))
        thr = jax.lax.bitcast_convert_type(prefix, jnp.float32)
        v = rs_vm[...]
        gt = v > thr
        cnt_gt = jnp.sum(gt.astype(jnp.int32))
        sum_gt = jnp.sum(jnp.where(gt, v, 0.0))
        train_bg = sum_gt + (k - cnt_gt).astype(jnp.float32) * thr
        bg_total = jnp.sum(v)
        fg_sum = tot_sum[0] - bg_total
        out_ref[0] = (fg_sum + train_bg) * 0.25


def kernel(logits, targets):
    N, C = logits.shape
    t2 = targets.reshape(N // 128, 128)
    B = 64
    nblk = (N // 128) // B
    out = pl.pallas_call(
        _ohem_kernel,
        grid=(nblk,),
        in_specs=[
            pl.BlockSpec((B * 128, C), lambda i: (i, 0)),
            pl.BlockSpec((B, 128), lambda i: (i, 0)),
        ],
        out_specs=pl.BlockSpec(memory_space=pltpu.SMEM),
        out_shape=jax.ShapeDtypeStruct((1,), jnp.float32),
        scratch_shapes=[
            pltpu.VMEM((N // 128, 128), jnp.float32),
            pltpu.SMEM((1,), jnp.float32),
            pltpu.SMEM((1,), jnp.int32),
            pltpu.SMEM((1,), jnp.int32),
        ],
        compiler_params=pltpu.CompilerParams(
            dimension_semantics=("arbitrary",),
        ),
    )(logits, t2)
    return out[0]


# speedup vs baseline: 1.8070x; 1.0410x over previous
import jax
import jax.numpy as jnp
from jax.experimental import pallas as pl
from jax.experimental.pallas import tpu as pltpu

_C = 80


def _ohem_kernel(x_ref, t_ref, out_ref, rs_vm, tot_sum, nr_fg):
    i = pl.program_id(0)
    nblk = pl.num_programs(0)
    B = t_ref.shape[0]

    @pl.when(i == 0)
    def _init():
        tot_sum[0] = jnp.float32(0.0)
        nr_fg[0] = jnp.int32(0)

    x = x_ref[...].reshape(B, 128, _C)
    t = t_ref[...]
    cls = jax.lax.broadcasted_iota(jnp.int32, (1, 1, _C), 2) + 1
    z = t[:, :, None] == cls
    _LOG2E = 1.4426950408889634
    _LN2 = 0.6931471805599453
    w = jnp.where(z, jnp.float32(-_LOG2E), jnp.float32(_LOG2E))
    le = jnp.log2(1.0 + jnp.exp2(x * w)) * jnp.float32(_LN2)
    rs = jnp.sum(le, axis=2)
    is_bg = t == 0
    rs_vm[pl.ds(i * B, B), :] = jnp.where(is_bg, rs, 0.0)
    tot_sum[0] += jnp.sum(rs)
    nr_fg[0] += jnp.sum((t > 0).astype(jnp.int32))

    @pl.when(i == nblk - 1)
    def _finish():
        n_total = nblk * B * 128
        k = jnp.maximum(jnp.int32(128),
                        jnp.minimum(n_total - nr_fg[0], nr_fg[0] * 3))

        def body(j, prefix):
            cand = prefix | jax.lax.shift_left(jnp.int32(1), 30 - j)
            candf = jax.lax.bitcast_convert_type(cand, jnp.float32)
            cnt = jnp.sum((rs_vm[...] >= candf).astype(jnp.int32))
            return jnp.where(cnt >= k, cand, prefix)

        prefix = jax.lax.fori_loop(0, 31, body, jnp.int32(0))
        thr = jax.lax.bitcast_convert_type(prefix, jnp.float32)
        v = rs_vm[...]
        gt = v > thr
        cnt_gt = jnp.sum(gt.astype(jnp.int32))
        sum_gt = jnp.sum(jnp.where(gt, v, 0.0))
        train_bg = sum_gt + (k - cnt_gt).astype(jnp.float32) * thr
        bg_total = jnp.sum(v)
        fg_sum = tot_sum[0] - bg_total
        out_ref[0] = (fg_sum + train_bg) * 0.25


def kernel(logits, targets):
    N, C = logits.shape
    t2 = targets.reshape(N // 128, 128)
    B = 64
    nblk = (N // 128) // B
    out = pl.pallas_call(
        _ohem_kernel,
        grid=(nblk,),
        in_specs=[
            pl.BlockSpec((B * 128, C), lambda i: (i, 0)),
            pl.BlockSpec((B, 128), lambda i: (i, 0)),
        ],
        out_specs=pl.BlockSpec(memory_space=pltpu.SMEM),
        out_shape=jax.ShapeDtypeStruct((1,), jnp.float32),
        scratch_shapes=[
            pltpu.VMEM((N // 128, 128), jnp.float32),
            pltpu.SMEM((1,), jnp.float32),
            pltpu.SMEM((1,), jnp.int32),
        ],
        compiler_params=pltpu.CompilerParams(
            dimension_semantics=("arbitrary",),
        ),
    )(logits, t2)
    return out[0]


# speedup vs baseline: 3.7374x; 2.0682x over previous
import jax
import jax.numpy as jnp
from jax.experimental import pallas as pl
from jax.experimental.pallas import tpu as pltpu

_C = 80


def _ohem_kernel(x_ref, t_ref, out_ref, rs_vm, tot_sum, nr_fg):
    i = pl.program_id(0)
    nblk = pl.num_programs(0)
    B = t_ref.shape[0]

    @pl.when(i == 0)
    def _init():
        tot_sum[0] = jnp.float32(0.0)
        nr_fg[0] = jnp.int32(0)

    x = x_ref[...]
    t = t_ref[...]
    cls = jax.lax.broadcasted_iota(jnp.int32, (_C, B), 0) + 1
    z = t[None, :] == cls
    _LOG2E = 1.4426950408889634
    _LN2 = 0.6931471805599453
    w = jnp.where(z, jnp.float32(-_LOG2E), jnp.float32(_LOG2E))
    le = jnp.log2(1.0 + jnp.exp2(x * w))
    rs = jnp.sum(le, axis=0) * jnp.float32(_LN2)
    is_bg = t == 0
    rs_vm[pl.ds(i * B, B)] = jnp.where(is_bg, rs, 0.0)
    tot_sum[0] += jnp.sum(rs)
    nr_fg[0] += jnp.sum((t > 0).astype(jnp.int32))

    @pl.when(i == nblk - 1)
    def _finish():
        n_total = nblk * B
        k = jnp.maximum(jnp.int32(128),
                        jnp.minimum(n_total - nr_fg[0], nr_fg[0] * 3))

        def body(j, prefix):
            cand = prefix | jax.lax.shift_left(jnp.int32(1), 30 - j)
            candf = jax.lax.bitcast_convert_type(cand, jnp.float32)
            cnt = jnp.sum((rs_vm[...] >= candf).astype(jnp.int32))
            return jnp.where(cnt >= k, cand, prefix)

        prefix = jax.lax.fori_loop(0, 31, body, jnp.int32(0))
        thr = jax.lax.bitcast_convert_type(prefix, jnp.float32)
        v = rs_vm[...]
        gt = v > thr
        cnt_gt = jnp.sum(gt.astype(jnp.int32))
        sum_gt = jnp.sum(jnp.where(gt, v, 0.0))
        train_bg = sum_gt + (k - cnt_gt).astype(jnp.float32) * thr
        bg_total = jnp.sum(v)
        fg_sum = tot_sum[0] - bg_total
        out_ref[0] = (fg_sum + train_bg) * 0.25


def kernel(logits, targets):
    N, C = logits.shape
    xt = logits.T
    B = 8192
    nblk = N // B
    out = pl.pallas_call(
        _ohem_kernel,
        grid=(nblk,),
        in_specs=[
            pl.BlockSpec((C, B), lambda i: (0, i)),
            pl.BlockSpec((B,), lambda i: (i,)),
        ],
        out_specs=pl.BlockSpec(memory_space=pltpu.SMEM),
        out_shape=jax.ShapeDtypeStruct((1,), jnp.float32),
        scratch_shapes=[
            pltpu.VMEM((N,), jnp.float32),
            pltpu.SMEM((1,), jnp.float32),
            pltpu.SMEM((1,), jnp.int32),
        ],
        compiler_params=pltpu.CompilerParams(
            dimension_semantics=("arbitrary",),
        ),
    )(xt, targets)
    return out[0]


# speedup vs baseline: 5.2559x; 1.4063x over previous
import jax
import jax.numpy as jnp
from jax.experimental import pallas as pl
from jax.experimental.pallas import tpu as pltpu

_C = 80


def _ohem_kernel(x_ref, t_ref, out_ref, rs_vm, tot_sum, nr_fg):
    i = pl.program_id(0)
    nblk = pl.num_programs(0)
    B = t_ref.shape[0]

    @pl.when(i == 0)
    def _init():
        tot_sum[0] = jnp.float32(0.0)
        nr_fg[0] = jnp.int32(0)

    x = x_ref[...]
    t = t_ref[...]
    cls = jax.lax.broadcasted_iota(jnp.int32, (_C, B), 0) + 1
    z = t[None, :] == cls
    _LOG2E = 1.4426950408889634
    _LN2 = 0.6931471805599453
    w = jnp.where(z, jnp.float32(-_LOG2E), jnp.float32(_LOG2E))
    le = jnp.log2(1.0 + jnp.exp2(x * w))
    rs = jnp.sum(le, axis=0) * jnp.float32(_LN2)
    is_bg = t == 0
    rs_vm[i, :] = jnp.where(is_bg, rs, 0.0)
    tot_sum[0] += jnp.sum(rs)
    nr_fg[0] += jnp.sum((t > 0).astype(jnp.int32))

    @pl.when(i == nblk - 1)
    def _finish():
        n_total = nblk * B
        k = jnp.maximum(jnp.int32(128),
                        jnp.minimum(n_total - nr_fg[0], nr_fg[0] * 3))

        def body(j, prefix):
            cand = prefix | jax.lax.shift_left(jnp.int32(1), 30 - j)
            candf = jax.lax.bitcast_convert_type(cand, jnp.float32)
            cnt = jnp.sum((rs_vm[...] >= candf).astype(jnp.int32))
            return jnp.where(cnt >= k, cand, prefix)

        prefix = jax.lax.fori_loop(0, 31, body, jnp.int32(0))
        thr = jax.lax.bitcast_convert_type(prefix, jnp.float32)
        v = rs_vm[...]
        gt = v > thr
        cnt_gt = jnp.sum(gt.astype(jnp.int32))
        sum_gt = jnp.sum(jnp.where(gt, v, 0.0))
        train_bg = sum_gt + (k - cnt_gt).astype(jnp.float32) * thr
        bg_total = jnp.sum(v)
        fg_sum = tot_sum[0] - bg_total
        out_ref[0] = (fg_sum + train_bg) * 0.25


def kernel(logits, targets):
    N, C = logits.shape
    xt = logits.T
    B = 8192
    nblk = N // B
    out = pl.pallas_call(
        _ohem_kernel,
        grid=(nblk,),
        in_specs=[
            pl.BlockSpec((C, B), lambda i: (0, i)),
            pl.BlockSpec((B,), lambda i: (i,)),
        ],
        out_specs=pl.BlockSpec(memory_space=pltpu.SMEM),
        out_shape=jax.ShapeDtypeStruct((1,), jnp.float32),
        scratch_shapes=[
            pltpu.VMEM((N // B, B), jnp.float32),
            pltpu.SMEM((1,), jnp.float32),
            pltpu.SMEM((1,), jnp.int32),
        ],
        compiler_params=pltpu.CompilerParams(
            dimension_semantics=("arbitrary",),
        ),
    )(xt, targets)
    return out[0]


# speedup vs baseline: 5.6652x; 1.0779x over previous
import jax
import jax.numpy as jnp
from jax.experimental import pallas as pl
from jax.experimental.pallas import tpu as pltpu

_C = 80


def _ohem_kernel(x_ref, t_ref, out_ref, rs_vm, tot_sum, nr_fg):
    i = pl.program_id(0)
    nblk = pl.num_programs(0)
    B = t_ref.shape[0]

    @pl.when(i == 0)
    def _init():
        tot_sum[0] = jnp.float32(0.0)
        nr_fg[0] = jnp.int32(0)

    x = x_ref[...]
    t = t_ref[...]
    cls = jax.lax.broadcasted_iota(jnp.int32, (_C, B), 0) + 1
    z = t[None, :] == cls
    _LOG2E = 1.4426950408889634
    _LN2 = 0.6931471805599453
    w = jnp.where(z, jnp.float32(-_LOG2E), jnp.float32(_LOG2E))
    le = jnp.log2(1.0 + jnp.exp2(x * w))
    rs = jnp.sum(le, axis=0) * jnp.float32(_LN2)
    is_bg = t == 0
    rs_vm[i, :] = jnp.where(is_bg, rs, 0.0)
    tot_sum[0] += jnp.sum(rs)
    nr_fg[0] += jnp.sum((t > 0).astype(jnp.int32))

    @pl.when(i == nblk - 1)
    def _finish():
        n_total = nblk * B
        k = jnp.maximum(jnp.int32(128),
                        jnp.minimum(n_total - nr_fg[0], nr_fg[0] * 3))

        def body(j, prefix):
            cand = prefix | jax.lax.shift_left(jnp.int32(1), 30 - j)
            candf = jax.lax.bitcast_convert_type(cand, jnp.float32)
            cnt = jnp.sum((rs_vm[...] >= candf).astype(jnp.int32))
            return jnp.where(cnt >= k, cand, prefix)

        prefix = jax.lax.fori_loop(0, 31, body, jnp.int32(0))
        thr = jax.lax.bitcast_convert_type(prefix, jnp.float32)
        v = rs_vm[...]
        gt = v > thr
        cnt_gt = jnp.sum(gt.astype(jnp.int32))
        sum_gt = jnp.sum(jnp.where(gt, v, 0.0))
        train_bg = sum_gt + (k - cnt_gt).astype(jnp.float32) * thr
        bg_total = jnp.sum(v)
        fg_sum = tot_sum[0] - bg_total
        out_ref[0] = (fg_sum + train_bg) * 0.25


def kernel(logits, targets):
    N, C = logits.shape
    xt = logits.T
    B = 16384
    nblk = N // B
    out = pl.pallas_call(
        _ohem_kernel,
        grid=(nblk,),
        in_specs=[
            pl.BlockSpec((C, B), lambda i: (0, i)),
            pl.BlockSpec((B,), lambda i: (i,)),
        ],
        out_specs=pl.BlockSpec(memory_space=pltpu.SMEM),
        out_shape=jax.ShapeDtypeStruct((1,), jnp.float32),
        scratch_shapes=[
            pltpu.VMEM((N // B, B), jnp.float32),
            pltpu.SMEM((1,), jnp.float32),
            pltpu.SMEM((1,), jnp.int32),
        ],
        compiler_params=pltpu.CompilerParams(
            dimension_semantics=("arbitrary",),
        ),
    )(xt, targets)
    return out[0]
